# baseline (speedup 1.0000x reference)
import functools

import jax
import jax.numpy as jnp
from jax import lax
from jax.experimental import pallas as pl
from jax.experimental.pallas import tpu as pltpu
from jax.experimental.pallas import tpu_sc as plsc

DIM = 128
BATCH = 16384
LANES = 128
NUM_CORES = 2
NUM_SUBCORES = 16
NW = NUM_CORES * NUM_SUBCORES
B_PER_W = BATCH // NW
CHUNKS = B_PER_W // LANES

_mesh = plsc.VectorSubcoreMesh(core_axis_name="c", subcore_axis_name="s")

TOT_CHUNKS = 2 * CHUNKS
SUPER = 2 * LANES
NSUPER = TOT_CHUNKS // 2


@functools.partial(
    pl.kernel,
    mesh=_mesh,
    out_type=[
        jax.ShapeDtypeStruct((BATCH, DIM), jnp.float32),
        jax.ShapeDtypeStruct((BATCH, DIM), jnp.float32),
    ],
    scratch_types=[
        pltpu.VMEM((TOT_CHUNKS, LANES), jnp.int32),
        pltpu.VMEM((SUPER, DIM), jnp.float32),
        pltpu.VMEM((SUPER, DIM), jnp.float32),
        *[pltpu.SemaphoreType.DMA for _ in range(6)],
    ],
)
def _sc_gather(doc_idx, neg_idx, table, dc_out, nc_out, idx_v, buf_a, buf_b, *sems):
    sem_g = sems[0:2]
    sem_o = sems[2:4]
    sem_id, sem_in = sems[4], sems[5]
    bufs = (buf_a, buf_b)
    wid = lax.axis_index("s") * NUM_CORES + lax.axis_index("c")
    base_row = wid * CHUNKS
    base_out = wid * B_PER_W
    idx_cp = [
        pltpu.async_copy(
            doc_idx.at[pl.ds(base_row, CHUNKS)], idx_v.at[pl.ds(0, CHUNKS)], sem_id
        ),
        pltpu.async_copy(
            neg_idx.at[pl.ds(base_row, CHUNKS)], idx_v.at[pl.ds(CHUNKS, CHUNKS)], sem_in
        ),
    ]
    dests = [(dc_out, base_out), (dc_out, base_out + SUPER),
             (nc_out, base_out), (nc_out, base_out + SUPER)]
    outs = [None] * NSUPER
    for s in range(NSUPER):
        b = s % 2
        if s == 0:
            idx_cp[0].wait()
        if s == NSUPER // 2:
            idx_cp[1].wait()
        if s >= 2:
            outs[s - 2].wait()
        g0 = pltpu.async_copy(
            table.at[idx_v.at[2 * s]], bufs[b].at[pl.ds(0, LANES)], sem_g[b]
        )
        g1 = pltpu.async_copy(
            table.at[idx_v.at[2 * s + 1]], bufs[b].at[pl.ds(LANES, LANES)], sem_g[b]
        )
        g0.wait()
        g1.wait()
        out_hbm, off = dests[s]
        outs[s] = pltpu.async_copy(bufs[b], out_hbm.at[pl.ds(off, SUPER)], sem_o[b])
    outs[NSUPER - 2].wait()
    outs[NSUPER - 1].wait()


def kernel(doc_center_ids, neg_center_ids, center_vecs):
    doc2 = doc_center_ids.reshape(BATCH // LANES, LANES)
    neg2 = neg_center_ids.reshape(BATCH // LANES, LANES)
    dc_emb, nc_emb = _sc_gather(doc2, neg2, center_vecs)
    return dc_emb, nc_emb


# speedup vs baseline: 1.0100x; 1.0100x over previous
import functools

import jax
import jax.numpy as jnp
from jax import lax
from jax.experimental import pallas as pl
from jax.experimental.pallas import tpu as pltpu
from jax.experimental.pallas import tpu_sc as plsc

DIM = 128
BATCH = 16384
LANES = 128
NUM_CORES = 2
NUM_SUBCORES = 16
NW = NUM_CORES * NUM_SUBCORES
B_PER_W = BATCH // NW
CHUNKS = B_PER_W // LANES

_mesh = plsc.VectorSubcoreMesh(core_axis_name="c", subcore_axis_name="s")

NBUF = 6
TOT_CHUNKS = 2 * CHUNKS


@functools.partial(
    pl.kernel,
    mesh=_mesh,
    out_type=[
        jax.ShapeDtypeStruct((BATCH, DIM), jnp.float32),
        jax.ShapeDtypeStruct((BATCH, DIM), jnp.float32),
    ],
    scratch_types=[
        pltpu.VMEM((TOT_CHUNKS, LANES), jnp.int32),
        *[pltpu.VMEM((LANES, DIM), jnp.float32) for _ in range(NBUF)],
        *[pltpu.SemaphoreType.DMA for _ in range(2 * NBUF + 2)],
    ],
)
def _sc_gather(doc_idx, neg_idx, table, dc_out, nc_out, idx_v, *scratch):
    bufs = scratch[:NBUF]
    sem_g = scratch[NBUF:2 * NBUF]
    sem_o = scratch[2 * NBUF:3 * NBUF]
    sem_id, sem_in = scratch[-2], scratch[-1]
    wid = lax.axis_index("s") * NUM_CORES + lax.axis_index("c")
    base_row = wid * CHUNKS
    base_out = wid * B_PER_W
    idx_cp = [
        pltpu.async_copy(
            doc_idx.at[pl.ds(base_row, CHUNKS)], idx_v.at[pl.ds(0, CHUNKS)], sem_id
        ),
        pltpu.async_copy(
            neg_idx.at[pl.ds(base_row, CHUNKS)], idx_v.at[pl.ds(CHUNKS, CHUNKS)], sem_in
        ),
    ]
    dests = [(dc_out, base_out + j * LANES) for j in range(CHUNKS)] + [
        (nc_out, base_out + j * LANES) for j in range(CHUNKS)
    ]
    gathers = [None] * TOT_CHUNKS
    outs = [None] * TOT_CHUNKS
    for i in range(TOT_CHUNKS):
        b = i % NBUF
        if i == 0:
            idx_cp[0].wait()
        if i == CHUNKS:
            idx_cp[1].wait()
        if i >= NBUF:
            outs[i - NBUF].wait()
        gathers[i] = pltpu.async_copy(table.at[idx_v.at[i]], bufs[b], sem_g[b])
        if i >= 1:
            p, pb = i - 1, (i - 1) % NBUF
            gathers[p].wait()
            out_hbm, off = dests[p]
            outs[p] = pltpu.async_copy(bufs[pb], out_hbm.at[pl.ds(off, LANES)], sem_o[pb])
    last = TOT_CHUNKS - 1
    gathers[last].wait()
    out_hbm, off = dests[last]
    outs[last] = pltpu.async_copy(bufs[last % NBUF], out_hbm.at[pl.ds(off, LANES)], sem_o[last % NBUF])
    for p in range(TOT_CHUNKS - NBUF, TOT_CHUNKS):
        outs[p].wait()


def kernel(doc_center_ids, neg_center_ids, center_vecs):
    doc2 = doc_center_ids.reshape(BATCH // LANES, LANES)
    neg2 = neg_center_ids.reshape(BATCH // LANES, LANES)
    dc_emb, nc_emb = _sc_gather(doc2, neg2, center_vecs)
    return dc_emb, nc_emb


# speedup vs baseline: 1.0308x; 1.0206x over previous
import functools

import jax
import jax.numpy as jnp
from jax import lax
from jax.experimental import pallas as pl
from jax.experimental.pallas import tpu as pltpu
from jax.experimental.pallas import tpu_sc as plsc

DIM = 128
BATCH = 16384
LANES = 128
NUM_CORES = 2
NUM_SUBCORES = 16
NW = NUM_CORES * NUM_SUBCORES
B_PER_W = BATCH // NW
CHUNKS = B_PER_W // LANES

_mesh = plsc.VectorSubcoreMesh(core_axis_name="c", subcore_axis_name="s")

NBUF = 4
TOT_CHUNKS = 2 * CHUNKS


@functools.partial(
    pl.kernel,
    mesh=_mesh,
    out_type=[
        jax.ShapeDtypeStruct((BATCH, DIM), jnp.float32),
        jax.ShapeDtypeStruct((BATCH, DIM), jnp.float32),
    ],
    scratch_types=[
        pltpu.VMEM((TOT_CHUNKS, LANES), jnp.int32),
        *[pltpu.VMEM((LANES, DIM), jnp.float32) for _ in range(NBUF)],
        *[pltpu.SemaphoreType.DMA for _ in range(2 * NBUF + 2)],
    ],
)
def _sc_gather(doc_idx, neg_idx, table, dc_out, nc_out, idx_v, *scratch):
    bufs = scratch[:NBUF]
    sem_g = scratch[NBUF:2 * NBUF]
    sem_o = scratch[2 * NBUF:3 * NBUF]
    sem_id, sem_in = scratch[-2], scratch[-1]
    wid = lax.axis_index("s") * NUM_CORES + lax.axis_index("c")
    base_row = wid * CHUNKS
    base_out = wid * B_PER_W
    idx_cp = [
        pltpu.async_copy(
            doc_idx.at[pl.ds(base_row, CHUNKS)], idx_v.at[pl.ds(0, CHUNKS)], sem_id
        ),
        pltpu.async_copy(
            neg_idx.at[pl.ds(base_row, CHUNKS)], idx_v.at[pl.ds(CHUNKS, CHUNKS)], sem_in
        ),
    ]
    dests = [(dc_out, base_out + j * LANES) for j in range(CHUNKS)] + [
        (nc_out, base_out + j * LANES) for j in range(CHUNKS)
    ]
    gathers = [None] * TOT_CHUNKS
    outs = [None] * TOT_CHUNKS
    for i in range(TOT_CHUNKS):
        b = i % NBUF
        if i == 0:
            idx_cp[0].wait()
        if i == CHUNKS:
            idx_cp[1].wait()
        if i >= NBUF:
            outs[i - NBUF].wait()
        gathers[i] = pltpu.async_copy(table.at[idx_v.at[i]], bufs[b], sem_g[b])
        if i >= 1:
            p, pb = i - 1, (i - 1) % NBUF
            gathers[p].wait()
            out_hbm, off = dests[p]
            outs[p] = pltpu.async_copy(bufs[pb], out_hbm.at[pl.ds(off, LANES)], sem_o[pb])
    last = TOT_CHUNKS - 1
    gathers[last].wait()
    out_hbm, off = dests[last]
    outs[last] = pltpu.async_copy(bufs[last % NBUF], out_hbm.at[pl.ds(off, LANES)], sem_o[last % NBUF])
    for p in range(TOT_CHUNKS - NBUF, TOT_CHUNKS):
        outs[p].wait()


def kernel(doc_center_ids, neg_center_ids, center_vecs):
    doc2 = doc_center_ids.reshape(BATCH // LANES, LANES)
    neg2 = neg_center_ids.reshape(BATCH // LANES, LANES)
    dc_emb, nc_emb = _sc_gather(doc2, neg2, center_vecs)
    return dc_emb, nc_emb
